# baseline (speedup 1.0000x reference)
import functools

import jax
import jax.numpy as jnp
from jax import lax
from jax.experimental import pallas as pl
from jax.experimental.pallas import tpu as pltpu
from jax.experimental.pallas import tpu_sc as plsc

_NC = 2
_NS = 16
_NW = _NC * _NS
_CHUNK = 128


def _sc_gather(user_table, item_table, uid4, iid4, s):
    n_slices, _, n_chunks, _ = uid4.shape
    D = user_table.shape[1]
    b_per_w = n_chunks * _CHUNK
    Bs = b_per_w * _NW
    n_jobs = 2 * n_chunks
    NB = min(4, n_jobs)
    mesh = plsc.VectorSubcoreMesh(core_axis_name="c", subcore_axis_name="s")

    @functools.partial(
        pl.kernel,
        mesh=mesh,
        out_type=[
            jax.ShapeDtypeStruct((Bs, D), jnp.float32),
            jax.ShapeDtypeStruct((Bs, D), jnp.float32),
        ],
        scratch_types=[
            pltpu.VMEM((n_chunks, _CHUNK), jnp.int32),
            pltpu.VMEM((n_chunks, _CHUNK), jnp.int32),
        ]
        + [pltpu.VMEM((_CHUNK, D), jnp.float32) for _ in range(NB)]
        + [pltpu.SemaphoreType.DMA, pltpu.SemaphoreType.DMA],
    )
    def gather_kernel(ut, it, uid, iid, u_out, i_out, idx_u, idx_i, *rest):
        bufs, (sem_g, sem_w) = rest[:NB], rest[NB:]
        wid = lax.axis_index("s") * _NC + lax.axis_index("c")
        base = wid * b_per_w
        pltpu.sync_copy(uid.at[s].at[wid], idx_u)
        pltpu.sync_copy(iid.at[s].at[wid], idx_i)
        jobs = []
        for idxv, tab, out in ((idx_u, ut, u_out), (idx_i, it, i_out)):
            for c in range(n_chunks):
                jobs.append((idxv.at[c], tab, out, base + c * _CHUNK))
        g = [None] * n_jobs
        w = [None] * n_jobs
        for c in range(min(NB, n_jobs)):
            idxr, tab, out, off = jobs[c]
            g[c] = pltpu.async_copy(tab.at[idxr], bufs[c % NB], sem_g)
        for c in range(n_jobs):
            g[c].wait()
            _, _, out, off = jobs[c]
            w[c] = pltpu.async_copy(bufs[c % NB], out.at[pl.ds(off, _CHUNK)], sem_w)
            nxt = c + NB
            if nxt < n_jobs:
                w[c].wait()
                idxr, tab, _, _ = jobs[nxt]
                g[nxt] = pltpu.async_copy(tab.at[idxr], bufs[nxt % NB], sem_g)
        for c in range(max(0, n_jobs - NB), n_jobs):
            w[c].wait()

    return gather_kernel(user_table, item_table, uid4, iid4)


def _mlp_body(u_ref, i_ref, w1a, w1b, b1r, w2, b2r, w3c, b3r, o_ref):
    h = jnp.dot(u_ref[...], w1a[...], preferred_element_type=jnp.float32)
    h += jnp.dot(i_ref[...], w1b[...], preferred_element_type=jnp.float32)
    h = jnp.maximum(h + b1r[...], 0.0)
    h2 = jnp.dot(h, w2[...], preferred_element_type=jnp.float32)
    h2 = jnp.maximum(h2 + b2r[...], 0.0)
    o = jnp.dot(h2, w3c[...], preferred_element_type=jnp.float32) + b3r[0, 0]
    o_ref[0] = jnp.transpose(1.0 / (1.0 + jnp.exp(-o)))


def _tc_mlp(u, i, W1, b1, W2, b2, W3, b3):
    B, D = u.shape
    H1 = W1.shape[1]
    H2 = W2.shape[1]
    blk = 4096
    grid = B // blk
    w1a = W1[:D]
    w1b = W1[D:]
    b1r = b1.reshape(1, H1)
    b2r = b2.reshape(1, H2)
    b3r = b3.reshape(1, 1)

    out = pl.pallas_call(
        _mlp_body,
        grid=(grid,),
        in_specs=[
            pl.BlockSpec((blk, D), lambda g: (g, 0)),
            pl.BlockSpec((blk, D), lambda g: (g, 0)),
            pl.BlockSpec((D, H1), lambda g: (0, 0)),
            pl.BlockSpec((D, H1), lambda g: (0, 0)),
            pl.BlockSpec((1, H1), lambda g: (0, 0)),
            pl.BlockSpec((H1, H2), lambda g: (0, 0)),
            pl.BlockSpec((1, H2), lambda g: (0, 0)),
            pl.BlockSpec((H2, 1), lambda g: (0, 0)),
            pl.BlockSpec((1, 1), lambda g: (0, 0)),
        ],
        out_specs=pl.BlockSpec((1, 1, blk), lambda g: (g, 0, 0)),
        out_shape=jax.ShapeDtypeStruct((grid, 1, blk), jnp.float32),
    )(u, i, w1a, w1b, b1r, W2, b2r, W3, b3r)
    return out.reshape(B)


def kernel(user_ids, item_ids, user_table, item_table, W1, b1, W2, b2, W3, b3):
    user_ids = user_ids.astype(jnp.int32)
    item_ids = item_ids.astype(jnp.int32)
    B = user_ids.shape[0]
    S = 2
    n_chunks = B // (S * _NW * _CHUNK)
    uid4 = user_ids.reshape(S, _NW, n_chunks, _CHUNK)
    iid4 = item_ids.reshape(S, _NW, n_chunks, _CHUNK)
    outs = []
    for s in range(S):
        u_emb, i_emb = _sc_gather(user_table, item_table, uid4, iid4, s)
        outs.append(_tc_mlp(u_emb, i_emb, W1, b1, W2, b2, W3, b3))
    return jnp.concatenate(outs)


# speedup vs baseline: 1.1028x; 1.1028x over previous
import functools

import jax
import jax.numpy as jnp
from jax import lax
from jax.experimental import pallas as pl
from jax.experimental.pallas import tpu as pltpu
from jax.experimental.pallas import tpu_sc as plsc

_NC = 2
_NS = 16
_NW = _NC * _NS
_CHUNK = 128


def _sc_gather(user_table, item_table, uid4, iid4, s):
    n_slices, _, n_chunks, _ = uid4.shape
    D = user_table.shape[1]
    b_per_w = n_chunks * _CHUNK
    Bs = b_per_w * _NW
    n_jobs = 2 * n_chunks
    NB = min(4, n_jobs)
    mesh = plsc.VectorSubcoreMesh(core_axis_name="c", subcore_axis_name="s")

    @functools.partial(
        pl.kernel,
        mesh=mesh,
        out_type=[
            jax.ShapeDtypeStruct((Bs, D), jnp.float32),
            jax.ShapeDtypeStruct((Bs, D), jnp.float32),
        ],
        scratch_types=[
            pltpu.VMEM((n_chunks, _CHUNK), jnp.int32),
            pltpu.VMEM((n_chunks, _CHUNK), jnp.int32),
        ]
        + [pltpu.VMEM((_CHUNK, D), jnp.float32) for _ in range(NB)]
        + [pltpu.SemaphoreType.DMA, pltpu.SemaphoreType.DMA],
    )
    def gather_kernel(ut, it, uid, iid, u_out, i_out, idx_u, idx_i, *rest):
        bufs, (sem_g, sem_w) = rest[:NB], rest[NB:]
        wid = lax.axis_index("s") * _NC + lax.axis_index("c")
        base = wid * b_per_w
        pltpu.sync_copy(uid.at[s].at[wid], idx_u)
        pltpu.sync_copy(iid.at[s].at[wid], idx_i)
        jobs = []
        for idxv, tab, out in ((idx_u, ut, u_out), (idx_i, it, i_out)):
            for c in range(n_chunks):
                jobs.append((idxv.at[c], tab, out, base + c * _CHUNK))
        g = [None] * n_jobs
        w = [None] * n_jobs
        for c in range(min(NB, n_jobs)):
            idxr, tab, out, off = jobs[c]
            g[c] = pltpu.async_copy(tab.at[idxr], bufs[c % NB], sem_g)
        for c in range(n_jobs):
            g[c].wait()
            _, _, out, off = jobs[c]
            w[c] = pltpu.async_copy(bufs[c % NB], out.at[pl.ds(off, _CHUNK)], sem_w)
            nxt = c + NB
            if nxt < n_jobs:
                w[c].wait()
                idxr, tab, _, _ = jobs[nxt]
                g[nxt] = pltpu.async_copy(tab.at[idxr], bufs[nxt % NB], sem_g)
        for c in range(max(0, n_jobs - NB), n_jobs):
            w[c].wait()

    return gather_kernel(user_table, item_table, uid4, iid4)


def _mlp_body(u_ref, i_ref, w1a, w1b, b1r, w2, b2r, w3c, b3r, o_ref):
    h = jnp.dot(u_ref[...], w1a[...], preferred_element_type=jnp.float32)
    h += jnp.dot(i_ref[...], w1b[...], preferred_element_type=jnp.float32)
    h = jnp.maximum(h + b1r[...], 0.0)
    h2 = jnp.dot(h, w2[...], preferred_element_type=jnp.float32)
    h2 = jnp.maximum(h2 + b2r[...], 0.0)
    o = jnp.dot(h2, w3c[...], preferred_element_type=jnp.float32) + b3r[0, 0]
    o_ref[0] = jnp.transpose(1.0 / (1.0 + jnp.exp(-o)))


def _tc_mlp(u, i, W1, b1, W2, b2, W3, b3):
    B, D = u.shape
    H1 = W1.shape[1]
    H2 = W2.shape[1]
    blk = 4096
    grid = B // blk
    w1a = W1[:D]
    w1b = W1[D:]
    b1r = b1.reshape(1, H1)
    b2r = b2.reshape(1, H2)
    b3r = b3.reshape(1, 1)

    out = pl.pallas_call(
        _mlp_body,
        grid=(grid,),
        in_specs=[
            pl.BlockSpec((blk, D), lambda g: (g, 0)),
            pl.BlockSpec((blk, D), lambda g: (g, 0)),
            pl.BlockSpec((D, H1), lambda g: (0, 0)),
            pl.BlockSpec((D, H1), lambda g: (0, 0)),
            pl.BlockSpec((1, H1), lambda g: (0, 0)),
            pl.BlockSpec((H1, H2), lambda g: (0, 0)),
            pl.BlockSpec((1, H2), lambda g: (0, 0)),
            pl.BlockSpec((H2, 1), lambda g: (0, 0)),
            pl.BlockSpec((1, 1), lambda g: (0, 0)),
        ],
        out_specs=pl.BlockSpec((1, 1, blk), lambda g: (g, 0, 0)),
        out_shape=jax.ShapeDtypeStruct((grid, 1, blk), jnp.float32),
    )(u, i, w1a, w1b, b1r, W2, b2r, W3, b3r)
    return out.reshape(B)


def kernel(user_ids, item_ids, user_table, item_table, W1, b1, W2, b2, W3, b3):
    user_ids = user_ids.astype(jnp.int32)
    item_ids = item_ids.astype(jnp.int32)
    B = user_ids.shape[0]
    S = 1
    n_chunks = B // (S * _NW * _CHUNK)
    uid4 = user_ids.reshape(S, _NW, n_chunks, _CHUNK)
    iid4 = item_ids.reshape(S, _NW, n_chunks, _CHUNK)
    outs = []
    for s in range(S):
        u_emb, i_emb = _sc_gather(user_table, item_table, uid4, iid4, s)
        outs.append(_tc_mlp(u_emb, i_emb, W1, b1, W2, b2, W3, b3))
    return jnp.concatenate(outs)
